# Initial kernel scaffold; baseline (speedup 1.0000x reference)
import jax
import jax.numpy as jnp
from jax.experimental import pallas as pl


def kernel(z_0, z_1, edge_index):
    raise NotImplementedError("write your pallas kernel here")


# speedup vs baseline: 1.2630x; 1.2630x over previous
import functools

import jax
import jax.numpy as jnp
from jax import lax
from jax.experimental import pallas as pl
from jax.experimental.pallas import tpu as pltpu
from jax.experimental.pallas import tpu_sc as plsc

NC = 2
NS = 16
LANES = 16
NW = NC * NS


@functools.partial(jax.jit, static_argnums=(3, 4, 5))
def _build_and_run(z_0, z_1, eidx, E, D, chunk):
    n_per_w = E // NW
    n_chunks = n_per_w // chunk
    groups = chunk // LANES
    mesh = plsc.VectorSubcoreMesh(core_axis_name="c", subcore_axis_name="s")

    @functools.partial(
        pl.kernel,
        out_type=jax.ShapeDtypeStruct((E,), jnp.float32),
        mesh=mesh,
        scratch_types=[
            pltpu.VMEM((n_per_w,), jnp.int32),
            pltpu.VMEM((n_per_w,), jnp.int32),
            pltpu.VMEM((chunk, D), jnp.float32),
            pltpu.VMEM((chunk, D), jnp.float32),
            pltpu.VMEM((n_per_w,), jnp.float32),
            pltpu.SemaphoreType.DMA,
        ],
        compiler_params=pltpu.CompilerParams(needs_layout_passes=False),
    )
    def k(z0_hbm, z1_hbm, idx0_hbm, idx1_hbm, out_hbm,
          idx0_v, idx1_v, src_v, dst_v, out_v, sem):
        wid = lax.axis_index("s") * NC + lax.axis_index("c")
        base = wid * n_per_w
        pltpu.sync_copy(idx0_hbm.at[pl.ds(base, n_per_w)], idx0_v)
        pltpu.sync_copy(idx1_hbm.at[pl.ds(base, n_per_w)], idx1_v)

        lane_iota = jnp.arange(LANES, dtype=jnp.int32)
        zero16 = jnp.zeros((LANES,), jnp.float32)

        def chunk_body(kk, carry):
            off = kk * chunk
            cp0 = pltpu.async_copy(
                z0_hbm.at[idx0_v.at[pl.ds(off, chunk)]], src_v, sem)
            cp1 = pltpu.async_copy(
                z1_hbm.at[idx1_v.at[pl.ds(off, chunk)]], dst_v, sem)
            cp0.wait()
            cp1.wait()
            for g in range(groups):
                lanes = g * LANES + lane_iota

                def dbody(i, accs):
                    a0, a1, a2, a3 = accs
                    d0 = i * 4
                    c0 = jnp.full((LANES,), d0, jnp.int32)
                    c1 = jnp.full((LANES,), d0 + 1, jnp.int32)
                    c2 = jnp.full((LANES,), d0 + 2, jnp.int32)
                    c3 = jnp.full((LANES,), d0 + 3, jnp.int32)
                    a0 = a0 + (plsc.load_gather(src_v, [lanes, c0])
                               * plsc.load_gather(dst_v, [lanes, c0]))
                    a1 = a1 + (plsc.load_gather(src_v, [lanes, c1])
                               * plsc.load_gather(dst_v, [lanes, c1]))
                    a2 = a2 + (plsc.load_gather(src_v, [lanes, c2])
                               * plsc.load_gather(dst_v, [lanes, c2]))
                    a3 = a3 + (plsc.load_gather(src_v, [lanes, c3])
                               * plsc.load_gather(dst_v, [lanes, c3]))
                    return a0, a1, a2, a3

                a0, a1, a2, a3 = lax.fori_loop(
                    0, D // 4, dbody, (zero16, zero16, zero16, zero16))
                x = (a0 + a1) + (a2 + a3)
                y = 1.0 / (1.0 + jnp.exp(-x))
                out_v[pl.ds(off + g * LANES, LANES)] = y
            return carry

        lax.fori_loop(0, n_chunks, chunk_body, 0)
        pltpu.sync_copy(out_v, out_hbm.at[pl.ds(base, n_per_w)])

    return k(z_0, z_1, eidx[0], eidx[1])


def kernel(z_0, z_1, edge_index):
    E = edge_index.shape[1]
    D = z_0.shape[1]
    eidx = edge_index.astype(jnp.int32)
    return _build_and_run(z_0, z_1, eidx, E, D, 80)


# speedup vs baseline: 1.4661x; 1.1608x over previous
import functools

import jax
import jax.numpy as jnp
from jax import lax
from jax.experimental import pallas as pl
from jax.experimental.pallas import tpu as pltpu
from jax.experimental.pallas import tpu_sc as plsc

NC = 2
NS = 16
LANES = 16
NW = NC * NS


@functools.partial(jax.jit, static_argnums=(3, 4, 5))
def _build_and_run(z_0, z_1, eidx, E, D, chunk):
    n_per_w = E // NW
    n_chunks = n_per_w // chunk
    n_pairs = (n_chunks - 1) // 2
    groups = chunk // LANES
    mesh = plsc.VectorSubcoreMesh(core_axis_name="c", subcore_axis_name="s")

    @functools.partial(
        pl.kernel,
        out_type=jax.ShapeDtypeStruct((E,), jnp.float32),
        mesh=mesh,
        scratch_types=[
            pltpu.VMEM((n_per_w,), jnp.int32),
            pltpu.VMEM((n_per_w,), jnp.int32),
            pltpu.VMEM((chunk, D), jnp.float32),
            pltpu.VMEM((chunk, D), jnp.float32),
            pltpu.VMEM((chunk, D), jnp.float32),
            pltpu.VMEM((chunk, D), jnp.float32),
            pltpu.VMEM((n_per_w,), jnp.float32),
            pltpu.SemaphoreType.DMA,
            pltpu.SemaphoreType.DMA,
        ],
        compiler_params=pltpu.CompilerParams(needs_layout_passes=False),
    )
    def k(z0_hbm, z1_hbm, idx0_hbm, idx1_hbm, out_hbm,
          idx0_v, idx1_v, src0_v, dst0_v, src1_v, dst1_v, out_v, sem0, sem1):
        wid = lax.axis_index("s") * NC + lax.axis_index("c")
        base = wid * n_per_w
        pltpu.sync_copy(idx0_hbm.at[pl.ds(base, n_per_w)], idx0_v)
        pltpu.sync_copy(idx1_hbm.at[pl.ds(base, n_per_w)], idx1_v)

        lane_iota = jnp.arange(LANES, dtype=jnp.int32)
        zero16 = jnp.zeros((LANES,), jnp.float32)

        def gathers(kk, src_buf, dst_buf, sem):
            off = kk * chunk
            return (
                pltpu.make_async_copy(
                    z0_hbm.at[idx0_v.at[pl.ds(off, chunk)]], src_buf, sem),
                pltpu.make_async_copy(
                    z1_hbm.at[idx1_v.at[pl.ds(off, chunk)]], dst_buf, sem),
            )

        def issue(kk, src_buf, dst_buf, sem):
            for cp in gathers(kk, src_buf, dst_buf, sem):
                cp.start()

        def drain(kk, src_buf, dst_buf, sem):
            for cp in gathers(kk, src_buf, dst_buf, sem):
                cp.wait()

        def compute(kk, src_buf, dst_buf):
            off = kk * chunk
            for g in range(groups):
                lanes = g * LANES + lane_iota

                def dbody(i, accs):
                    a0, a1, a2, a3 = accs
                    d0 = i * 4
                    c0 = jnp.full((LANES,), d0, jnp.int32)
                    c1 = jnp.full((LANES,), d0 + 1, jnp.int32)
                    c2 = jnp.full((LANES,), d0 + 2, jnp.int32)
                    c3 = jnp.full((LANES,), d0 + 3, jnp.int32)
                    a0 = a0 + (plsc.load_gather(src_buf, [lanes, c0])
                               * plsc.load_gather(dst_buf, [lanes, c0]))
                    a1 = a1 + (plsc.load_gather(src_buf, [lanes, c1])
                               * plsc.load_gather(dst_buf, [lanes, c1]))
                    a2 = a2 + (plsc.load_gather(src_buf, [lanes, c2])
                               * plsc.load_gather(dst_buf, [lanes, c2]))
                    a3 = a3 + (plsc.load_gather(src_buf, [lanes, c3])
                               * plsc.load_gather(dst_buf, [lanes, c3]))
                    return a0, a1, a2, a3

                a0, a1, a2, a3 = lax.fori_loop(
                    0, D // 4, dbody, (zero16, zero16, zero16, zero16))
                x = (a0 + a1) + (a2 + a3)
                y = 1.0 / (1.0 + jnp.exp(-x))
                out_v[pl.ds(off + g * LANES, LANES)] = y

        issue(0, src0_v, dst0_v, sem0)

        def pair_body(m, carry):
            k0 = 2 * m
            drain(k0, src0_v, dst0_v, sem0)
            issue(k0 + 1, src1_v, dst1_v, sem1)
            compute(k0, src0_v, dst0_v)
            drain(k0 + 1, src1_v, dst1_v, sem1)
            issue(k0 + 2, src0_v, dst0_v, sem0)
            compute(k0 + 1, src1_v, dst1_v)
            return carry

        lax.fori_loop(0, n_pairs, pair_body, 0)

        klast = 2 * n_pairs
        drain(klast, src0_v, dst0_v, sem0)
        if klast + 1 < n_chunks:
            issue(klast + 1, src1_v, dst1_v, sem1)
        compute(klast, src0_v, dst0_v)
        if klast + 1 < n_chunks:
            drain(klast + 1, src1_v, dst1_v, sem1)
            compute(klast + 1, src1_v, dst1_v)

        pltpu.sync_copy(out_v, out_hbm.at[pl.ds(base, n_per_w)])

    return k(z_0, z_1, eidx[0], eidx[1])


def kernel(z_0, z_1, edge_index):
    E = edge_index.shape[1]
    D = z_0.shape[1]
    eidx = edge_index.astype(jnp.int32)
    return _build_and_run(z_0, z_1, eidx, E, D, 80)
